# baseline (speedup 1.0000x reference)
import functools

import jax
import jax.numpy as jnp
from jax import lax
from jax.experimental import pallas as pl
from jax.experimental.pallas import tpu as pltpu
from jax.experimental.pallas import tpu_sc as plsc

_C = 19
_K = 2048
_TAB = _C * _K
_SCALE = float(_K) * (1.0 - 2.0 ** -13)

_WBLK = 8192

_NC = 2
_NS = 16
_NW = _NC * _NS
_CHUNK = 2048


def _softmax_err_body(*refs):
    x_ref, lab_ref = refs[0], refs[1]
    outs = refs[2:]
    x = jnp.reshape(x_ref[0], (_C, _WBLK))
    lab = jnp.reshape(lab_ref[0], (1, _WBLK))
    m = jnp.max(x, axis=0, keepdims=True)
    e = jnp.exp(x - m)
    p = e / jnp.sum(e, axis=0, keepdims=True)
    cls = lax.broadcasted_iota(jnp.int32, (_C, _WBLK), 0)
    fg = cls == lab
    eh = jnp.where(fg, -(1.0 - p), p)
    for c in range(_C):
        outs[c][...] = eh[c]
    p_lab = jnp.sum(jnp.where(fg, p, 0.0), axis=0, keepdims=True)
    fgidx = lab * _K + ((1.0 - p_lab) * _SCALE).astype(jnp.int32)
    outs[_C][...] = jnp.reshape(fgidx, (_WBLK,))


def _softmax_err(x, lab, npix):
    b_dim, _, h, w = x.shape
    rows = _WBLK // w
    nblk = h // rows
    out_sds = tuple(
        jax.ShapeDtypeStruct((npix,), jnp.float32) for _ in range(_C)
    ) + (jax.ShapeDtypeStruct((npix,), jnp.int32),)
    out_specs = tuple(
        pl.BlockSpec((_WBLK,), lambda b, j: (b * nblk + j,))
        for _ in range(_C + 1))
    return pl.pallas_call(
        _softmax_err_body,
        grid=(b_dim, nblk),
        in_specs=[
            pl.BlockSpec((1, _C, rows, w), lambda b, j: (b, 0, j, 0)),
            pl.BlockSpec((1, rows, w), lambda b, j: (b, j, 0)),
        ],
        out_specs=out_specs,
        out_shape=out_sds,
    )(x, lab)


def _histogram_sc(ehats, npix):
    pix_per_tile = npix // _NW
    nchunk = pix_per_tile // _CHUNK
    assert nchunk % 2 == 0

    mesh = plsc.VectorSubcoreMesh(
        core_axis_name="c", subcore_axis_name="s",
        num_cores=_NC, num_subcores=_NS)

    @functools.partial(
        pl.kernel,
        out_type=jax.ShapeDtypeStruct((_NW * 2 * _TAB,), jnp.float32),
        mesh=mesh,
        compiler_params=pltpu.CompilerParams(needs_layout_passes=False),
        scratch_types=[
            pltpu.VMEM((_CHUNK,), jnp.float32),
            pltpu.VMEM((_CHUNK,), jnp.float32),
            pltpu.VMEM((_CHUNK,), jnp.int32),
            pltpu.VMEM((_CHUNK,), jnp.int32),
            pltpu.VMEM((_TAB,), jnp.float32),
            pltpu.VMEM((_TAB,), jnp.float32),
            pltpu.SemaphoreType.DMA,
            pltpu.SemaphoreType.DMA,
        ],
    )
    def hist(*refs):
        ehat = refs[:_C]
        fgstream = refs[_C]
        out_hbm = refs[_C + 1]
        buf0, buf1, ibuf0, ibuf1, tabn, tabf, sem0, sem1 = refs[_C + 2:]

        wid = lax.axis_index("s") * _NC + lax.axis_index("c")
        base = wid * pix_per_tile

        zeros = jnp.zeros((16,), jnp.float32)

        def zinit(i, carry):
            tabn[pl.ds(i * 16, 16)] = zeros
            tabf[pl.ds(i * 16, 16)] = zeros
            return carry
        lax.fori_loop(0, _TAB // 16, zinit, 0, unroll=4)

        ones = jnp.ones((16,), jnp.float32)

        batch = 8

        def process(buf, coff):
            def vbody(i, inner):
                ehs = [buf[pl.ds((i * batch + t) * 16, 16)]
                       for t in range(batch)]
                idxs = [(jnp.abs(eh) * _SCALE).astype(jnp.int32) + coff
                        for eh in ehs]
                for t in range(batch):
                    plsc.addupdate_scatter(tabn, [idxs[t]], ones)
                return inner
            lax.fori_loop(0, _CHUNK // (16 * batch), vbody, 0)

        def process_fg(buf):
            def vbody(i, inner):
                idxs = [buf[pl.ds((i * batch + t) * 16, 16)]
                        for t in range(batch)]
                for t in range(batch):
                    plsc.addupdate_scatter(tabf, [idxs[t]], ones)
                return inner
            lax.fori_loop(0, _CHUNK // (16 * batch), vbody, 0)

        def stream_one(src, b0, b1, proc):
            pltpu.async_copy(src.at[pl.ds(base, _CHUNK)], b0, sem0)

            def pair(j, carry):
                pltpu.async_copy(
                    src.at[pl.ds(base + (j + 1) * _CHUNK, _CHUNK)],
                    b1, sem1)
                pltpu.make_async_copy(
                    src.at[pl.ds(base, _CHUNK)], b0, sem0).wait()
                proc(b0)

                @pl.when(j + 2 < nchunk)
                def _():
                    pltpu.async_copy(
                        src.at[pl.ds(base + (j + 2) * _CHUNK, _CHUNK)],
                        b0, sem0)
                pltpu.make_async_copy(
                    src.at[pl.ds(base, _CHUNK)], b1, sem1).wait()
                proc(b1)
                return carry
            lax.fori_loop(0, nchunk // 2, lambda j, c2: pair(j * 2, c2), 0)

        for c in range(_C):
            coff = c * _K
            stream_one(ehat[c], buf0, buf1,
                       lambda b, coff=coff: process(b, coff))
        stream_one(fgstream, ibuf0, ibuf1, process_fg)

        obase = wid * (2 * _TAB)
        pltpu.sync_copy(tabn, out_hbm.at[pl.ds(obase, _TAB)])
        pltpu.sync_copy(tabf, out_hbm.at[pl.ds(obase + _TAB, _TAB)])

    return hist(*ehats)


def _cumsum_lane(x):
    n = x.shape[-1]
    s = 1
    while s < n:
        shifted = jnp.concatenate(
            [jnp.zeros(x.shape[:-1] + (s,), x.dtype), x[..., :-s]], axis=-1)
        x = x + shifted
        s *= 2
    return x


def _finish_body(tab_ref, out_ref):
    t = jnp.sum(tab_ref[...], axis=0)
    n = t[0]
    f = t[1]
    b = n - f
    ftot = jnp.sum(f, axis=1, keepdims=True)
    btot = jnp.sum(b, axis=1, keepdims=True)
    icf = _cumsum_lane(f)
    icb = _cumsum_lane(b)
    cf = ftot - icf + f
    cb = btot - icb + b
    denom = jnp.maximum(ftot + cb, 1.0)
    jac = 1.0 - (ftot - cf) / denom
    jac_prev = jnp.concatenate(
        [jac[:, 1:], jnp.zeros((_C, 1), jnp.float32)], axis=1)
    djac = jac - jac_prev
    ebar = (lax.broadcasted_iota(jnp.int32, (_C, _K), 1).astype(
        jnp.float32) + 0.5) * (1.0 / _SCALE)
    loss_c = jnp.sum(ebar * djac, axis=1, keepdims=True)
    pres = (ftot > 0.0).astype(jnp.float32)
    loss = jnp.sum(loss_c * pres) / jnp.maximum(jnp.sum(pres), 1.0)
    out_ref[...] = jnp.reshape(loss, (1, 1))


def _finish(tabs):
    return pl.pallas_call(
        _finish_body,
        out_shape=jax.ShapeDtypeStruct((1, 1), jnp.float32),
    )(tabs)


def kernel(output, target):
    b_dim, c_dim, h, w = output.shape
    npix = b_dim * h * w
    ehats = _softmax_err(output, target, npix)
    tabs = _histogram_sc(ehats, npix)
    loss = _finish(tabs.reshape(_NW, 2, _C, _K))
    return loss.reshape(())


# speedup vs baseline: 1.1037x; 1.1037x over previous
import functools

import jax
import jax.numpy as jnp
from jax import lax
from jax.experimental import pallas as pl
from jax.experimental.pallas import tpu as pltpu
from jax.experimental.pallas import tpu_sc as plsc

_C = 19
_K = 2048
_TAB = _C * _K
_SCALE = float(_K) * (1.0 - 2.0 ** -13)

_WBLK = 8192

_NC = 2
_NS = 16
_NW = _NC * _NS
_CHUNK = 2048


def _softmax_err_body(*refs):
    x_ref, lab_ref = refs[0], refs[1]
    outs = refs[2:]
    x = jnp.reshape(x_ref[0], (_C, _WBLK))
    lab = jnp.reshape(lab_ref[0], (1, _WBLK))
    m = jnp.max(x, axis=0, keepdims=True)
    e = jnp.exp(x - m)
    p = e / jnp.sum(e, axis=0, keepdims=True)
    cls = lax.broadcasted_iota(jnp.int32, (_C, _WBLK), 0)
    fg = cls == lab
    eh = jnp.where(fg, -(1.0 - p), p)
    for c in range(_C):
        outs[c][...] = eh[c]


def _softmax_err(x, lab, row_off, nrows):
    b_dim, _, h, w = x.shape
    rows = _WBLK // w
    nblk = nrows // rows
    joff = row_off // rows
    npart = b_dim * nrows * w
    out_sds = tuple(
        jax.ShapeDtypeStruct((npart,), jnp.float32) for _ in range(_C))
    out_specs = tuple(
        pl.BlockSpec((_WBLK,), lambda b, j: (b * nblk + j,))
        for _ in range(_C))
    return pl.pallas_call(
        _softmax_err_body,
        grid=(b_dim, nblk),
        in_specs=[
            pl.BlockSpec((1, _C, rows, w), lambda b, j: (b, 0, j + joff, 0)),
            pl.BlockSpec((1, rows, w), lambda b, j: (b, j + joff, 0)),
        ],
        out_specs=out_specs,
        out_shape=out_sds,
    )(x, lab)


def _histogram_sc(ehats, npix):
    pix_per_tile = npix // _NW
    nchunk = pix_per_tile // _CHUNK
    assert nchunk % 2 == 0

    mesh = plsc.VectorSubcoreMesh(
        core_axis_name="c", subcore_axis_name="s",
        num_cores=_NC, num_subcores=_NS)

    @functools.partial(
        pl.kernel,
        out_type=jax.ShapeDtypeStruct((_NW * 2 * _TAB,), jnp.float32),
        mesh=mesh,
        compiler_params=pltpu.CompilerParams(needs_layout_passes=False),
        scratch_types=[
            pltpu.VMEM((_CHUNK,), jnp.float32),
            pltpu.VMEM((_CHUNK,), jnp.float32),
            pltpu.VMEM((_TAB,), jnp.float32),
            pltpu.VMEM((_TAB,), jnp.float32),
            pltpu.SemaphoreType.DMA,
            pltpu.SemaphoreType.DMA,
        ],
    )
    def hist(*refs):
        ehat = refs[:_C]
        out_hbm = refs[_C]
        buf0, buf1, tabn, tabf, sem0, sem1 = refs[_C + 1:]

        wid = lax.axis_index("s") * _NC + lax.axis_index("c")
        base = wid * pix_per_tile

        zeros = jnp.zeros((16,), jnp.float32)

        def zinit(i, carry):
            tabn[pl.ds(i * 16, 16)] = zeros
            tabf[pl.ds(i * 16, 16)] = zeros
            return carry
        lax.fori_loop(0, _TAB // 16, zinit, 0, unroll=4)

        ones = jnp.ones((16,), jnp.float32)

        batch = 8

        def process(buf, coff):
            def vbody(i, inner):
                ehs = [buf[pl.ds((i * batch + t) * 16, 16)]
                       for t in range(batch)]
                fgs = [eh < 0.0 for eh in ehs]
                idxs = [(jnp.abs(eh) * _SCALE).astype(jnp.int32) + coff
                        for eh in ehs]
                for t in range(batch):
                    plsc.addupdate_scatter(tabn, [idxs[t]], ones)
                    plsc.addupdate_scatter(tabf, [idxs[t]], ones,
                                           mask=fgs[t])
                return inner
            lax.fori_loop(0, _CHUNK // (16 * batch), vbody, 0)

        for c in range(_C):
            src = ehat[c]
            coff = c * _K
            pltpu.async_copy(src.at[pl.ds(base, _CHUNK)], buf0, sem0)

            def pair(j, carry, src=src, coff=coff):
                pltpu.async_copy(
                    src.at[pl.ds(base + (j + 1) * _CHUNK, _CHUNK)],
                    buf1, sem1)
                pltpu.make_async_copy(
                    src.at[pl.ds(base, _CHUNK)], buf0, sem0).wait()
                process(buf0, coff)

                @pl.when(j + 2 < nchunk)
                def _():
                    pltpu.async_copy(
                        src.at[pl.ds(base + (j + 2) * _CHUNK, _CHUNK)],
                        buf0, sem0)
                pltpu.make_async_copy(
                    src.at[pl.ds(base, _CHUNK)], buf1, sem1).wait()
                process(buf1, coff)
                return carry
            lax.fori_loop(0, nchunk // 2, lambda j, c2: pair(j * 2, c2), 0)

        obase = wid * (2 * _TAB)
        pltpu.sync_copy(tabn, out_hbm.at[pl.ds(obase, _TAB)])
        pltpu.sync_copy(tabf, out_hbm.at[pl.ds(obase + _TAB, _TAB)])

    return hist(*ehats)


def _cumsum_lane(x):
    n = x.shape[-1]
    s = 1
    while s < n:
        shifted = jnp.concatenate(
            [jnp.zeros(x.shape[:-1] + (s,), x.dtype), x[..., :-s]], axis=-1)
        x = x + shifted
        s *= 2
    return x


def _finish_body(tab0_ref, tab1_ref, out_ref):
    t = jnp.sum(tab0_ref[...], axis=0) + jnp.sum(tab1_ref[...], axis=0)
    n = t[0]
    f = t[1]
    b = n - f
    ftot = jnp.sum(f, axis=1, keepdims=True)
    btot = jnp.sum(b, axis=1, keepdims=True)
    icf = _cumsum_lane(f)
    icb = _cumsum_lane(b)
    cf = ftot - icf + f
    cb = btot - icb + b
    denom = jnp.maximum(ftot + cb, 1.0)
    jac = 1.0 - (ftot - cf) / denom
    jac_prev = jnp.concatenate(
        [jac[:, 1:], jnp.zeros((_C, 1), jnp.float32)], axis=1)
    djac = jac - jac_prev
    ebar = (lax.broadcasted_iota(jnp.int32, (_C, _K), 1).astype(
        jnp.float32) + 0.5) * (1.0 / _SCALE)
    loss_c = jnp.sum(ebar * djac, axis=1, keepdims=True)
    pres = (ftot > 0.0).astype(jnp.float32)
    loss = jnp.sum(loss_c * pres) / jnp.maximum(jnp.sum(pres), 1.0)
    out_ref[...] = jnp.reshape(loss, (1, 1))


def _finish(tabs0, tabs1):
    return pl.pallas_call(
        _finish_body,
        out_shape=jax.ShapeDtypeStruct((1, 1), jnp.float32),
    )(tabs0, tabs1)


def kernel(output, target):
    b_dim, c_dim, h, w = output.shape
    half = h // 2
    npart = b_dim * half * w
    ehats0 = _softmax_err(output, target, 0, half)
    tabs0 = _histogram_sc(ehats0, npart)
    ehats1 = _softmax_err(output, target, half, half)
    tabs1 = _histogram_sc(ehats1, npart)
    loss = _finish(tabs0.reshape(_NW, 2, _C, _K),
                   tabs1.reshape(_NW, 2, _C, _K))
    return loss.reshape(())


# speedup vs baseline: 1.1566x; 1.0480x over previous
import functools

import jax
import jax.numpy as jnp
from jax import lax
from jax.experimental import pallas as pl
from jax.experimental.pallas import tpu as pltpu
from jax.experimental.pallas import tpu_sc as plsc

_C = 19
_K = 2048
_TAB = _C * _K
_SCALE = float(_K) * (1.0 - 2.0 ** -13)

_WBLK = 8192

_NC = 2
_NS = 16
_NW = _NC * _NS
_CHUNK = 2048


def _softmax_err_body(*refs):
    x_ref, lab_ref = refs[0], refs[1]
    outs = refs[2:]
    x = jnp.reshape(x_ref[0], (_C, _WBLK))
    lab = jnp.reshape(lab_ref[0], (1, _WBLK))
    m = jnp.max(x, axis=0, keepdims=True)
    e = jnp.exp(x - m)
    p = e / jnp.sum(e, axis=0, keepdims=True)
    cls = lax.broadcasted_iota(jnp.int32, (_C, _WBLK), 0)
    fg = cls == lab
    eh = jnp.where(fg, -(1.0 - p), p)
    for c in range(_C):
        outs[c][...] = eh[c]


def _softmax_err(x, lab, row_off, nrows):
    b_dim, _, h, w = x.shape
    rows = _WBLK // w
    nblk = nrows // rows
    joff = row_off // rows
    npart = b_dim * nrows * w
    out_sds = tuple(
        jax.ShapeDtypeStruct((npart,), jnp.float32) for _ in range(_C))
    out_specs = tuple(
        pl.BlockSpec((_WBLK,), lambda b, j: (b * nblk + j,))
        for _ in range(_C))
    return pl.pallas_call(
        _softmax_err_body,
        grid=(b_dim, nblk),
        in_specs=[
            pl.BlockSpec((1, _C, rows, w), lambda b, j: (b, 0, j + joff, 0)),
            pl.BlockSpec((1, rows, w), lambda b, j: (b, j + joff, 0)),
        ],
        out_specs=out_specs,
        out_shape=out_sds,
    )(x, lab)


def _histogram_sc(ehats, npix):
    pix_per_tile = npix // _NW
    nchunk = pix_per_tile // _CHUNK
    assert nchunk % 2 == 0

    mesh = plsc.VectorSubcoreMesh(
        core_axis_name="c", subcore_axis_name="s",
        num_cores=_NC, num_subcores=_NS)

    @functools.partial(
        pl.kernel,
        out_type=jax.ShapeDtypeStruct((_NW * 2 * _TAB,), jnp.float32),
        mesh=mesh,
        compiler_params=pltpu.CompilerParams(needs_layout_passes=False),
        scratch_types=[
            pltpu.VMEM((_CHUNK,), jnp.float32),
            pltpu.VMEM((_CHUNK,), jnp.float32),
            pltpu.VMEM((_TAB,), jnp.float32),
            pltpu.VMEM((_TAB,), jnp.float32),
            pltpu.SemaphoreType.DMA,
            pltpu.SemaphoreType.DMA,
        ],
    )
    def hist(*refs):
        ehat = refs[:_C]
        out_hbm = refs[_C]
        buf0, buf1, tabn, tabf, sem0, sem1 = refs[_C + 1:]

        wid = lax.axis_index("s") * _NC + lax.axis_index("c")
        base = wid * pix_per_tile

        zeros = jnp.zeros((16,), jnp.float32)

        def zinit(i, carry):
            tabn[pl.ds(i * 16, 16)] = zeros
            tabf[pl.ds(i * 16, 16)] = zeros
            return carry
        lax.fori_loop(0, _TAB // 16, zinit, 0, unroll=4)

        ones = jnp.ones((16,), jnp.float32)

        batch = 8

        def process(buf, coff):
            def vbody(i, inner):
                ehs = [buf[pl.ds((i * batch + t) * 16, 16)]
                       for t in range(batch)]
                fgs = [eh < 0.0 for eh in ehs]
                idxs = [(jnp.abs(eh) * _SCALE).astype(jnp.int32) + coff
                        for eh in ehs]
                for t in range(batch):
                    plsc.addupdate_scatter(tabn, [idxs[t]], ones)
                    plsc.addupdate_scatter(tabf, [idxs[t]], ones,
                                           mask=fgs[t])
                return inner
            lax.fori_loop(0, _CHUNK // (16 * batch), vbody, 0)

        for c in range(_C):
            src = ehat[c]
            coff = c * _K
            pltpu.async_copy(src.at[pl.ds(base, _CHUNK)], buf0, sem0)

            def pair(j, carry, src=src, coff=coff):
                pltpu.async_copy(
                    src.at[pl.ds(base + (j + 1) * _CHUNK, _CHUNK)],
                    buf1, sem1)
                pltpu.make_async_copy(
                    src.at[pl.ds(base, _CHUNK)], buf0, sem0).wait()
                process(buf0, coff)

                @pl.when(j + 2 < nchunk)
                def _():
                    pltpu.async_copy(
                        src.at[pl.ds(base + (j + 2) * _CHUNK, _CHUNK)],
                        buf0, sem0)
                pltpu.make_async_copy(
                    src.at[pl.ds(base, _CHUNK)], buf1, sem1).wait()
                process(buf1, coff)
                return carry
            lax.fori_loop(0, nchunk // 2, lambda j, c2: pair(j * 2, c2), 0)

        obase = wid * (2 * _TAB)
        pltpu.sync_copy(tabn, out_hbm.at[pl.ds(obase, _TAB)])
        pltpu.sync_copy(tabf, out_hbm.at[pl.ds(obase + _TAB, _TAB)])

    return hist(*ehats)


def _cumsum_lane(x):
    n = x.shape[-1]
    s = 1
    while s < n:
        shifted = jnp.concatenate(
            [jnp.zeros(x.shape[:-1] + (s,), x.dtype), x[..., :-s]], axis=-1)
        x = x + shifted
        s *= 2
    return x


def _finish_body(tab0_ref, tab1_ref, out_ref):
    accn = jnp.zeros((_TAB,), jnp.float32)
    accf = jnp.zeros((_TAB,), jnp.float32)
    for r in (tab0_ref, tab1_ref):
        for wimg in range(_NW):
            accn = accn + r[pl.ds(wimg * 2 * _TAB, _TAB)]
            accf = accf + r[pl.ds(wimg * 2 * _TAB + _TAB, _TAB)]
    n = jnp.reshape(accn, (_C, _K))
    f = jnp.reshape(accf, (_C, _K))
    b = n - f
    ftot = jnp.sum(f, axis=1, keepdims=True)
    btot = jnp.sum(b, axis=1, keepdims=True)
    icf = _cumsum_lane(f)
    icb = _cumsum_lane(b)
    cf = ftot - icf + f
    cb = btot - icb + b
    denom = jnp.maximum(ftot + cb, 1.0)
    jac = 1.0 - (ftot - cf) / denom
    jac_prev = jnp.concatenate(
        [jac[:, 1:], jnp.zeros((_C, 1), jnp.float32)], axis=1)
    djac = jac - jac_prev
    ebar = (lax.broadcasted_iota(jnp.int32, (_C, _K), 1).astype(
        jnp.float32) + 0.5) * (1.0 / _SCALE)
    loss_c = jnp.sum(ebar * djac, axis=1, keepdims=True)
    pres = (ftot > 0.0).astype(jnp.float32)
    loss = jnp.sum(loss_c * pres) / jnp.maximum(jnp.sum(pres), 1.0)
    out_ref[...] = jnp.reshape(loss, (1, 1))


def _finish(tabs0, tabs1):
    return pl.pallas_call(
        _finish_body,
        out_shape=jax.ShapeDtypeStruct((1, 1), jnp.float32),
    )(tabs0, tabs1)


def kernel(output, target):
    b_dim, c_dim, h, w = output.shape
    half = h // 2
    npart = b_dim * half * w
    ehats0 = _softmax_err(output, target, 0, half)
    tabs0 = _histogram_sc(ehats0, npart)
    ehats1 = _softmax_err(output, target, half, half)
    tabs1 = _histogram_sc(ehats1, npart)
    loss = _finish(tabs0, tabs1)
    return loss.reshape(())
